# baseline (speedup 1.0000x reference)
import functools

import jax
import jax.numpy as jnp
from jax import lax
from jax.experimental import pallas as pl
from jax.experimental.pallas import tpu as pltpu
from jax.experimental.pallas import tpu_sc as plsc

_U = 50000
_E = 800000
_D = 64
_B = 16384
_HALF = 25024
_NH = 25088
_NPG = 2 * _NH
_EPAD = 802816
_IPT = _EPAD // 16
_ISB = 3584
_NISB = _IPT // _ISB
_CAP = 28672
_PC = 64
_CROWS = _CAP // _PC
_PROWS = 2 * 16 * _CROWS
_SBR = 32
_NSB = _CROWS // _SBR
_BLK = 64
_NBLK = _SBR
_NBODY = _NBLK // 4
_RPT = _NH // 16
_RFULL = _RPT // 64
_RREM = _RPT - _RFULL * 64
_BPT = _B // 32


@functools.lru_cache(maxsize=None)
def _partition():
    mesh = plsc.VectorSubcoreMesh(core_axis_name="c", subcore_axis_name="s")

    @functools.partial(
        pl.kernel,
        mesh=mesh,
        compiler_params=pltpu.CompilerParams(
            needs_layout_passes=False, use_tc_tiling_on_sc=False),
        out_type=(jax.ShapeDtypeStruct((2 * 16 * _CAP,), jnp.int32),
                  jax.ShapeDtypeStruct((2 * 16 * _CAP,), jnp.int32),
                  jax.ShapeDtypeStruct((2 * 16 * _CAP,), jnp.float32)),
        scratch_types=[
            pltpu.VMEM((_CAP,), jnp.int32),
            pltpu.VMEM((_CAP,), jnp.int32),
            pltpu.VMEM((_CAP,), jnp.float32),
            pltpu.VMEM((_ISB,), jnp.int32),
            pltpu.VMEM((_ISB,), jnp.int32),
            pltpu.VMEM((_ISB,), jnp.float32),
        ],
    )
    def part(src, dst, val, osrc, odst, oval, sa, da, va, ss, ds_, vs):
        c = lax.axis_index("c")
        s = lax.axis_index("s")

        def zf(i, carry):
            sa[pl.ds(i * 16, 16)] = jnp.zeros((16,), jnp.int32)
            da[pl.ds(i * 16, 16)] = jnp.zeros((16,), jnp.int32)
            va[pl.ds(i * 16, 16)] = jnp.zeros((16,), jnp.float32)
            return carry

        lax.fori_loop(0, _CAP // 16, zf, 0)

        lo = c * _HALF
        base = s * _IPT

        def sb_body(sb, fill):
            off = base + sb * _ISB
            pltpu.sync_copy(src.at[pl.ds(off, _ISB)], ss)
            pltpu.sync_copy(dst.at[pl.ds(off, _ISB)], ds_)
            pltpu.sync_copy(val.at[pl.ds(off, _ISB)], vs)

            def grp(g, fill2):
                o = g * 16
                s16 = ss[pl.ds(o, 16)]
                d16 = ds_[pl.ds(o, 16)]
                v16 = vs[pl.ds(o, 16)]
                m = (d16 >= lo) & (d16 < lo + _HALF)
                ld = d16 - lo
                plsc.store_compressed(sa.at[pl.ds(fill2, 16)], s16, mask=m)
                plsc.store_compressed(da.at[pl.ds(fill2, 16)], ld, mask=m)
                plsc.store_compressed(va.at[pl.ds(fill2, 16)], v16, mask=m)
                cnt = plsc.all_reduce_population_count(m)
                return fill2 + cnt[0]

            return lax.fori_loop(0, _ISB // 16, grp, fill)

        lax.fori_loop(0, _NISB, sb_body, jnp.int32(0))

        rb = (c * 16 + s) * _CAP
        pltpu.sync_copy(sa, osrc.at[pl.ds(rb, _CAP)])
        pltpu.sync_copy(da, odst.at[pl.ds(rb, _CAP)])
        pltpu.sync_copy(va, oval.at[pl.ds(rb, _CAP)])

    return part


@functools.lru_cache(maxsize=None)
def _spmm2():
    mesh = plsc.VectorSubcoreMesh(core_axis_name="c", subcore_axis_name="s")

    @functools.partial(
        pl.kernel,
        mesh=mesh,
        compiler_params=pltpu.CompilerParams(
            needs_layout_passes=False, use_tc_tiling_on_sc=False),
        out_type=jax.ShapeDtypeStruct((_NPG, _D), jnp.float32),
        scratch_types=[
            pltpu.VMEM_SHARED((_NH, _D), jnp.float32),
            pltpu.VMEM((_SBR, _PC), jnp.int32),
            pltpu.VMEM((_SBR, _PC), jnp.int32),
            pltpu.VMEM((_SBR, _PC), jnp.float32),
            pltpu.VMEM((_BLK, _D), jnp.float32),
            pltpu.VMEM((_BLK, _D), jnp.float32),
            pltpu.VMEM((_BLK, _D), jnp.float32),
            pltpu.VMEM((_BLK, _D), jnp.float32),
            pltpu.SemaphoreType.DMA,
            pltpu.SemaphoreType.DMA,
            pltpu.SemaphoreType.DMA,
            pltpu.SemaphoreType.DMA,
            pltpu.SemaphoreType.DMA,
            pltpu.SemaphoreType.DMA,
            pltpu.SemaphoreType.DMA,
            pltpu.SemaphoreType.DMA,
        ],
    )
    def spmm2(srcA, dstA, valA, xA, srcB, dstB, valB, xB, out,
              acc, sidx_sb, didx_sb, vals_sb, b0, b1, b2, b3,
              g0, g1, g2, g3, s0, s1, s2, s3):
        c = lax.axis_index("c")
        s = lax.axis_index("s")
        bufs = (b0, b1, b2, b3)
        gsem = (g0, g1, g2, g3)
        ssem = (s0, s1, s2, s3)

        def zrow(i, carry):
            for t in range(4):
                b0[i, pl.ds(t * 16, 16)] = jnp.zeros((16,), jnp.float32)
            return carry

        lax.fori_loop(0, _BLK, zrow, 0)

        def zcp(k, carry):
            pltpu.sync_copy(b0, acc.at[pl.ds(s * _RPT + k * _BLK, _BLK)])
            return carry

        lax.fori_loop(0, _RFULL, zcp, 0)
        pltpu.sync_copy(b0.at[pl.ds(0, _RREM)],
                        acc.at[pl.ds(s * _RPT + _RFULL * _BLK, _RREM)])
        plsc.subcore_barrier()

        def process(src2, dst2, val2, x2):
            rbase = (c * 16 + s) * _CROWS

            def gather_issue(q, i):
                pltpu.async_copy(x2.at[sidx_sb.at[q]], bufs[i], gsem[i])

            def gather_wait(q, i):
                pltpu.make_async_copy(
                    x2.at[sidx_sb.at[q]], bufs[i], gsem[i]).wait()

            def scat_issue(q, i):
                pltpu.async_copy(bufs[i], acc.at[didx_sb.at[q]],
                                 ssem[i], add=True)

            def scat_wait(q, i):
                pltpu.make_async_copy(
                    bufs[i], acc.at[didx_sb.at[q]], ssem[i]).wait()

            def scale(q, i):
                buf = bufs[i]

                def grp(g, carry):
                    v16 = vals_sb[q, pl.ds(g * 16, 16)]
                    for j in range(16):
                        e = g * 16 + j
                        sv = v16[j]
                        for t in range(4):
                            buf[e, pl.ds(t * 16, 16)] = (
                                buf[e, pl.ds(t * 16, 16)] * sv)
                    return carry

                lax.fori_loop(0, _BLK // 16, grp, 0)

            def super_body(sb, carry):
                r0 = rbase + sb * _SBR
                pltpu.sync_copy(src2.at[pl.ds(r0, _SBR)], sidx_sb)
                pltpu.sync_copy(dst2.at[pl.ds(r0, _SBR)], didx_sb)
                pltpu.sync_copy(val2.at[pl.ds(r0, _SBR)], vals_sb)
                gather_issue(0, 0)
                gather_issue(1, 1)

                def body(m, carry2):
                    for i in range(4):
                        q = 4 * m + i
                        j = (i + 2) % 4
                        gather_wait(q, i)
                        scale(q, i)
                        scat_issue(q, i)
                        if i < 2:
                            @pl.when(m > 0)
                            def _():
                                scat_wait(q - 2, j)

                            gather_issue(q + 2, j)
                        else:
                            @pl.when(m < _NBODY - 1)
                            def _():
                                scat_wait(q - 2, j)
                                gather_issue(q + 2, j)
                    return carry2

                lax.fori_loop(0, _NBODY, body, 0)
                for i in range(4):
                    scat_wait(_NBLK - 4 + i, i)
                return carry

            lax.fori_loop(0, _NSB, super_body, 0)

        process(srcA, dstA, valA, xA)
        process(srcB, dstB, valB, xB)
        plsc.subcore_barrier()

        def wb(k, carry):
            r0 = s * _RPT + k * _BLK
            pltpu.sync_copy(acc.at[pl.ds(r0, _BLK)],
                            out.at[pl.ds(c * _NH + r0, _BLK)])
            return carry

        lax.fori_loop(0, _RFULL, wb, 0)
        r0 = s * _RPT + _RFULL * _BLK
        pltpu.sync_copy(acc.at[pl.ds(r0, _RREM)],
                        out.at[pl.ds(c * _NH + r0, _RREM)])

    return spmm2


@functools.lru_cache(maxsize=None)
def _gamma():
    mesh = plsc.VectorSubcoreMesh(core_axis_name="c", subcore_axis_name="s")

    @functools.partial(
        pl.kernel,
        mesh=mesh,
        compiler_params=pltpu.CompilerParams(
            needs_layout_passes=False, use_tc_tiling_on_sc=False),
        out_type=jax.ShapeDtypeStruct((_B,), jnp.float32),
        scratch_types=[
            pltpu.VMEM((128,), jnp.int32),
            pltpu.VMEM((128,), jnp.int32),
            pltpu.VMEM((128, _D), jnp.float32),
            pltpu.VMEM((128, _D), jnp.float32),
            pltpu.VMEM((_BPT,), jnp.float32),
            pltpu.SemaphoreType.DMA,
            pltpu.SemaphoreType.DMA,
        ],
    )
    def gk(au, ai, uidx_h, iidx_h, out, uidx, iidx, ub, ib, gv, semu, semi):
        wid = lax.axis_index("s") * 2 + lax.axis_index("c")
        for kk in range(_BPT // 128):
            off = wid * _BPT + kk * 128
            pltpu.sync_copy(uidx_h.at[pl.ds(off, 128)], uidx)
            pltpu.sync_copy(iidx_h.at[pl.ds(off, 128)], iidx)
            cu = pltpu.async_copy(au.at[uidx], ub, semu)
            ci = pltpu.async_copy(ai.at[iidx], ib, semi)
            cu.wait()
            ci.wait()

            def grp(g, carry):
                rowi = g * 16 + lax.iota(jnp.int32, 16)
                acc = jnp.zeros((16,), jnp.float32)
                for dd in range(_D):
                    cold = jnp.full((16,), dd, jnp.int32)
                    acc = acc + (plsc.load_gather(ub, [rowi, cold])
                                 * plsc.load_gather(ib, [rowi, cold]))
                gv[pl.ds(kk * 128 + g * 16, 16)] = acc
                return carry

            lax.fori_loop(0, 8, grp, 0)
        pltpu.sync_copy(gv, out.at[pl.ds(wid * _BPT, _BPT)])

    return gk


def _tog(w):
    z = jnp.zeros((64, w.shape[1]), w.dtype)
    z2 = jnp.zeros((112, w.shape[1]), w.dtype)
    return jnp.concatenate([w[:_HALF], z, w[_HALF:], z2], axis=0)


def _gap_ids(ids):
    return ids + 64 * (ids >= _HALF).astype(jnp.int32)


def _prep_edges(part, src, dst, val):
    pz = _EPAD - _E
    srcp = jnp.concatenate([src, jnp.zeros((pz,), jnp.int32)])
    dstp = jnp.concatenate([dst, jnp.zeros((pz,), jnp.int32)])
    valp = jnp.concatenate([val, jnp.zeros((pz,), jnp.float32)])
    osrc, odst, oval = part(_gap_ids(srcp), dstp, valp)
    return (osrc.reshape(_PROWS, _PC), odst.reshape(_PROWS, _PC),
            oval.reshape(_PROWS, _PC))


def kernel(users, items, uv_src, uv_dst, uv_val, uu_src, uu_dst, uu_val,
           vv_src, vv_dst, vv_val, vu_src, vu_dst, vu_val,
           du, dv, user_emb_w, item_emb_w):
    part = _partition()
    spmm2 = _spmm2()
    gk = _gamma()
    uw2, iw2 = _tog(user_emb_w), _tog(item_emb_w)
    duo, dvo = _tog(du * user_emb_w), _tog(dv * item_emb_w)
    uu = _prep_edges(part, uu_src, uu_dst, uu_val)
    uv = _prep_edges(part, uv_src, uv_dst, uv_val)
    vv = _prep_edges(part, vv_src, vv_dst, vv_val)
    vu = _prep_edges(part, vu_src, vu_dst, vu_val)
    ue, ie = uw2, iw2
    su, si = uw2, iw2
    for _ in range(3):
        ueff, ieff = ue + duo, ie + dvo
        ue = spmm2(uu[0], uu[1], uu[2], ueff, uv[0], uv[1], uv[2], ieff)
        ie = spmm2(vv[0], vv[1], vv[2], ieff, vu[0], vu[1], vu[2], ue)
        su, si = su + ue, si + ie
    au, ai = su * 0.25, si * 0.25
    return gk(au, ai, _gap_ids(users), _gap_ids(items))


# speedup vs baseline: 7.8993x; 7.8993x over previous
import functools

import jax
import jax.numpy as jnp
from jax import lax
from jax.experimental import pallas as pl
from jax.experimental.pallas import tpu as pltpu
from jax.experimental.pallas import tpu_sc as plsc

_U = 50000
_V = 50000
_E = 800000
_H = 32
_B = 16384
_NW = 32
_NP = 50048
_CHUNK = 128
_EPAD = 802816
_ROWS = _EPAD // _CHUNK
_RPT_E = _ROWS // 16
_SBR = 28
_NSB = _RPT_E // _SBR
_BLK = 128
_TPB = _BLK // _CHUNK
_NBLK = _SBR * _CHUNK // _BLK
_NBODY = _NBLK // 4
_RPT = _NP // 16
_RFULL = _RPT // _CHUNK
_RREM = _RPT - _RFULL * _CHUNK
_BPT = _B // _NW


@functools.lru_cache(maxsize=None)
def _spmm2():
    mesh = plsc.VectorSubcoreMesh(core_axis_name="c", subcore_axis_name="s")

    @functools.partial(
        pl.kernel,
        mesh=mesh,
        compiler_params=pltpu.CompilerParams(
            needs_layout_passes=False, use_tc_tiling_on_sc=False),
        out_type=jax.ShapeDtypeStruct((2 * _NP, _H), jnp.float32),
        scratch_types=[
            pltpu.VMEM_SHARED((_NP, _H), jnp.float32),
            pltpu.VMEM((_SBR, _CHUNK), jnp.int32),
            pltpu.VMEM((_SBR, _CHUNK), jnp.int32),
            pltpu.VMEM((_SBR, _CHUNK), jnp.float32),
            pltpu.VMEM((_BLK, _H), jnp.float32),
            pltpu.VMEM((_BLK, _H), jnp.float32),
            pltpu.VMEM((_BLK, _H), jnp.float32),
            pltpu.VMEM((_BLK, _H), jnp.float32),
            pltpu.SemaphoreType.DMA,
            pltpu.SemaphoreType.DMA,
            pltpu.SemaphoreType.DMA,
            pltpu.SemaphoreType.DMA,
            pltpu.SemaphoreType.DMA,
            pltpu.SemaphoreType.DMA,
            pltpu.SemaphoreType.DMA,
            pltpu.SemaphoreType.DMA,
        ],
    )
    def spmm2(srcA, dstA, valA, xA, srcB, dstB, valB, xB, out,
              acc, sidx_sb, didx_sb, vals_sb, b0, b1, b2, b3,
              g0, g1, g2, g3, s0, s1, s2, s3):
        c = lax.axis_index("c")
        s = lax.axis_index("s")
        bufs = (b0, b1, b2, b3)
        gsem = (g0, g1, g2, g3)
        ssem = (s0, s1, s2, s3)

        def zrow(i, carry):
            b0[i, pl.ds(0, 16)] = jnp.zeros((16,), jnp.float32)
            b0[i, pl.ds(16, 16)] = jnp.zeros((16,), jnp.float32)
            return carry

        lax.fori_loop(0, _CHUNK, zrow, 0)

        def zcp(k, carry):
            pltpu.sync_copy(b0.at[pl.ds(0, _CHUNK)],
                            acc.at[pl.ds(s * _RPT + k * _CHUNK, _CHUNK)])
            return carry

        lax.fori_loop(0, _RFULL, zcp, 0)
        pltpu.sync_copy(b0.at[pl.ds(0, _RREM)],
                        acc.at[pl.ds(s * _RPT + _RFULL * _CHUNK, _RREM)])
        plsc.subcore_barrier()

        def process(src2, dst2, val2, x2):
            sbase = c * _ROWS + s * _RPT_E
            dbase = s * _RPT_E

            def gather_issue(q, i):
                for t in range(_TPB):
                    pltpu.async_copy(x2.at[sidx_sb.at[_TPB * q + t]],
                                     bufs[i].at[pl.ds(t * _CHUNK, _CHUNK)],
                                     gsem[i])

            def gather_wait(q, i):
                for t in range(_TPB):
                    pltpu.make_async_copy(
                        x2.at[sidx_sb.at[_TPB * q + t]],
                        bufs[i].at[pl.ds(t * _CHUNK, _CHUNK)],
                        gsem[i]).wait()

            def scat_issue(q, i):
                for t in range(_TPB):
                    pltpu.async_copy(bufs[i].at[pl.ds(t * _CHUNK, _CHUNK)],
                                     acc.at[didx_sb.at[_TPB * q + t]],
                                     ssem[i], add=True)

            def scat_wait(q, i):
                for t in range(_TPB):
                    pltpu.make_async_copy(
                        bufs[i].at[pl.ds(t * _CHUNK, _CHUNK)],
                        acc.at[didx_sb.at[_TPB * q + t]],
                        ssem[i]).wait()

            def scale(q, i):
                buf = bufs[i]

                def grp(g, carry):
                    row = _TPB * q + g // 8
                    lane0 = (g % 8) * 16
                    v16 = vals_sb[row, pl.ds(lane0, 16)]
                    for j in range(16):
                        e = g * 16 + j
                        sv = v16[j]
                        buf[e, pl.ds(0, 16)] = buf[e, pl.ds(0, 16)] * sv
                        buf[e, pl.ds(16, 16)] = buf[e, pl.ds(16, 16)] * sv
                    return carry

                lax.fori_loop(0, _BLK // 16, grp, 0)

            def super_body(sb, carry):
                pltpu.sync_copy(src2.at[pl.ds(sbase + sb * _SBR, _SBR)], sidx_sb)
                pltpu.sync_copy(dst2.at[pl.ds(dbase + sb * _SBR, _SBR)], didx_sb)
                pltpu.sync_copy(val2.at[pl.ds(dbase + sb * _SBR, _SBR)], vals_sb)
                gather_issue(0, 0)
                gather_issue(1, 1)

                def body(m, carry2):
                    for i in range(4):
                        q = 4 * m + i
                        j = (i + 2) % 4
                        gather_wait(q, i)
                        scale(q, i)
                        scat_issue(q, i)
                        if i < 2:
                            @pl.when(m > 0)
                            def _():
                                scat_wait(q - 2, j)

                            gather_issue(q + 2, j)
                        else:
                            @pl.when(m < _NBODY - 1)
                            def _():
                                scat_wait(q - 2, j)
                                gather_issue(q + 2, j)
                    return carry2

                lax.fori_loop(0, _NBODY, body, 0)
                for i in range(4):
                    scat_wait(_NBLK - 4 + i, i)
                return carry

            lax.fori_loop(0, _NSB, super_body, 0)

        process(srcA, dstA, valA, xA)
        process(srcB, dstB, valB, xB)
        plsc.subcore_barrier()

        def wb(k, carry):
            r0 = s * _RPT + k * _CHUNK
            pltpu.sync_copy(acc.at[pl.ds(r0, _CHUNK)],
                            out.at[pl.ds(c * _NP + r0, _CHUNK)])
            return carry

        lax.fori_loop(0, _RFULL, wb, 0)
        r0 = s * _RPT + _RFULL * _CHUNK
        pltpu.sync_copy(acc.at[pl.ds(r0, _RREM)],
                        out.at[pl.ds(c * _NP + r0, _RREM)])

    return spmm2


@functools.lru_cache(maxsize=None)
def _gamma():
    mesh = plsc.VectorSubcoreMesh(core_axis_name="c", subcore_axis_name="s")

    @functools.partial(
        pl.kernel,
        mesh=mesh,
        compiler_params=pltpu.CompilerParams(
            needs_layout_passes=False, use_tc_tiling_on_sc=False),
        out_type=jax.ShapeDtypeStruct((_B,), jnp.float32),
        scratch_types=[
            pltpu.VMEM((_CHUNK,), jnp.int32),
            pltpu.VMEM((_CHUNK,), jnp.int32),
            pltpu.VMEM((_CHUNK, _H), jnp.float32),
            pltpu.VMEM((_CHUNK, _H), jnp.float32),
            pltpu.VMEM((_CHUNK, _H), jnp.float32),
            pltpu.VMEM((_CHUNK, _H), jnp.float32),
            pltpu.VMEM((_BPT,), jnp.float32),
            pltpu.SemaphoreType.DMA,
        ],
    )
    def gk(au, ai, u0idx, u1idx, i0idx, i1idx, out,
           uidx, iidx, u0, u1, i0, i1, gv, sem):
        wid = lax.axis_index("s") * 2 + lax.axis_index("c")
        for kk in range(_BPT // _CHUNK):
            off = wid * _BPT + kk * _CHUNK
            pltpu.sync_copy(u0idx.at[pl.ds(off, _CHUNK)], uidx)
            pltpu.async_copy(au.at[uidx], u0, sem).wait()
            pltpu.sync_copy(u1idx.at[pl.ds(off, _CHUNK)], uidx)
            pltpu.async_copy(au.at[uidx], u1, sem).wait()
            pltpu.sync_copy(i0idx.at[pl.ds(off, _CHUNK)], iidx)
            pltpu.async_copy(ai.at[iidx], i0, sem).wait()
            pltpu.sync_copy(i1idx.at[pl.ds(off, _CHUNK)], iidx)
            pltpu.async_copy(ai.at[iidx], i1, sem).wait()

            def grp(g, carry):
                rowi = g * 16 + lax.iota(jnp.int32, 16)
                acc = jnp.zeros((16,), jnp.float32)
                for dd in range(_H):
                    cold = jnp.full((16,), dd, jnp.int32)
                    acc = acc + plsc.load_gather(u0, [rowi, cold]) * plsc.load_gather(i0, [rowi, cold])
                    acc = acc + plsc.load_gather(u1, [rowi, cold]) * plsc.load_gather(i1, [rowi, cold])
                gv[pl.ds(kk * _CHUNK + g * 16, 16)] = acc
                return carry

            lax.fori_loop(0, _CHUNK // 16, grp, 0)
        pltpu.sync_copy(gv, out.at[pl.ds(wid * _BPT, _BPT)])

    return gk


def _to2(w):
    wp = jnp.pad(w, ((0, _NP - w.shape[0]), (0, 0)))
    return jnp.concatenate([wp[:, :_H], wp[:, _H:]], axis=0)


def _pad_edges(src, dst, val):
    pz = _EPAD - _E
    srcp = jnp.concatenate([src, jnp.zeros((pz,), jnp.int32)])
    src2 = jnp.concatenate([srcp, srcp + _NP])
    dstp = jnp.concatenate([dst, jnp.zeros((pz,), jnp.int32)])
    valp = jnp.concatenate([val, jnp.zeros((pz,), jnp.float32)])
    return (src2.reshape(2 * _ROWS, _CHUNK),
            dstp.reshape(_ROWS, _CHUNK),
            valp.reshape(_ROWS, _CHUNK))


def kernel(users, items, uv_src, uv_dst, uv_val, uu_src, uu_dst, uu_val,
           vv_src, vv_dst, vv_val, vu_src, vu_dst, vu_val,
           du, dv, user_emb_w, item_emb_w):
    spmm2 = _spmm2()
    gk = _gamma()
    uw2, iw2 = _to2(user_emb_w), _to2(item_emb_w)
    duo, dvo = _to2(du * user_emb_w), _to2(dv * item_emb_w)
    uu = _pad_edges(uu_src, uu_dst, uu_val)
    uv = _pad_edges(uv_src, uv_dst, uv_val)
    vv = _pad_edges(vv_src, vv_dst, vv_val)
    vu = _pad_edges(vu_src, vu_dst, vu_val)
    ue, ie = uw2, iw2
    su, si = uw2, iw2
    for _ in range(3):
        ueff, ieff = ue + duo, ie + dvo
        ue = spmm2(uu[0], uu[1], uu[2], ueff, uv[0], uv[1], uv[2], ieff)
        ie = spmm2(vv[0], vv[1], vv[2], ieff, vu[0], vu[1], vu[2], ue)
        su, si = su + ue, si + ie
    au, ai = su * 0.25, si * 0.25
    return gk(au, ai, users, users + _NP, items, items + _NP)


# speedup vs baseline: 8.3021x; 1.0510x over previous
import functools

import jax
import jax.numpy as jnp
from jax import lax
from jax.experimental import pallas as pl
from jax.experimental.pallas import tpu as pltpu
from jax.experimental.pallas import tpu_sc as plsc

_U = 50000
_V = 50000
_E = 800000
_H = 32
_B = 16384
_NW = 32
_NP = 50048
_CHUNK = 128
_EPAD = 802816
_ROWS = _EPAD // _CHUNK
_RPT_E = _ROWS // 16
_SBR = 28
_NSB = _RPT_E // _SBR
_BLK = 128
_TPB = _BLK // _CHUNK
_NBLK = _SBR * _CHUNK // _BLK
_NBODY = _NBLK // 4
_RPT = _NP // 16
_RFULL = _RPT // _CHUNK
_RREM = _RPT - _RFULL * _CHUNK
_BPT = _B // _NW


@functools.lru_cache(maxsize=None)
def _spmm2():
    mesh = plsc.VectorSubcoreMesh(core_axis_name="c", subcore_axis_name="s")

    @functools.partial(
        pl.kernel,
        mesh=mesh,
        compiler_params=pltpu.CompilerParams(
            needs_layout_passes=False, use_tc_tiling_on_sc=False),
        out_type=jax.ShapeDtypeStruct((2 * _NP, _H), jnp.float32),
        scratch_types=[
            pltpu.VMEM_SHARED((_NP, _H), jnp.float32),
            pltpu.VMEM((_SBR, _CHUNK), jnp.int32),
            pltpu.VMEM((_SBR, _CHUNK), jnp.int32),
            pltpu.VMEM((_SBR, _CHUNK), jnp.float32),
            pltpu.VMEM((_BLK, _H), jnp.float32),
            pltpu.VMEM((_BLK, _H), jnp.float32),
            pltpu.VMEM((_BLK, _H), jnp.float32),
            pltpu.VMEM((_BLK, _H), jnp.float32),
            pltpu.SemaphoreType.DMA,
            pltpu.SemaphoreType.DMA,
            pltpu.SemaphoreType.DMA,
            pltpu.SemaphoreType.DMA,
            pltpu.SemaphoreType.DMA,
            pltpu.SemaphoreType.DMA,
            pltpu.SemaphoreType.DMA,
            pltpu.SemaphoreType.DMA,
        ],
    )
    def spmm2(srcA, dstA, valA, xA, srcB, dstB, valB, xB, out,
              acc, sidx_sb, didx_sb, vals_sb, b0, b1, b2, b3,
              g0, g1, g2, g3, s0, s1, s2, s3):
        c = lax.axis_index("c")
        s = lax.axis_index("s")
        bufs = (b0, b1, b2, b3)
        gsem = (g0, g1, g2, g3)
        ssem = (s0, s1, s2, s3)

        def zrow(i, carry):
            b0[i, pl.ds(0, 16)] = jnp.zeros((16,), jnp.float32)
            b0[i, pl.ds(16, 16)] = jnp.zeros((16,), jnp.float32)
            return carry

        lax.fori_loop(0, _CHUNK, zrow, 0)

        def zcp(k, carry):
            pltpu.sync_copy(b0.at[pl.ds(0, _CHUNK)],
                            acc.at[pl.ds(s * _RPT + k * _CHUNK, _CHUNK)])
            return carry

        lax.fori_loop(0, _RFULL, zcp, 0)
        pltpu.sync_copy(b0.at[pl.ds(0, _RREM)],
                        acc.at[pl.ds(s * _RPT + _RFULL * _CHUNK, _RREM)])
        plsc.subcore_barrier()

        def process(src2, dst2, val2, x2):
            sbase = c * _ROWS + s * _RPT_E
            dbase = s * _RPT_E

            def gather_issue(q, i):
                for t in range(_TPB):
                    pltpu.async_copy(x2.at[sidx_sb.at[_TPB * q + t]],
                                     bufs[i].at[pl.ds(t * _CHUNK, _CHUNK)],
                                     gsem[i])

            def gather_wait(q, i):
                for t in range(_TPB):
                    pltpu.make_async_copy(
                        x2.at[sidx_sb.at[_TPB * q + t]],
                        bufs[i].at[pl.ds(t * _CHUNK, _CHUNK)],
                        gsem[i]).wait()

            def scat_issue(q, i):
                for t in range(_TPB):
                    pltpu.async_copy(bufs[i].at[pl.ds(t * _CHUNK, _CHUNK)],
                                     acc.at[didx_sb.at[_TPB * q + t]],
                                     ssem[i], add=True)

            def scat_wait(q, i):
                for t in range(_TPB):
                    pltpu.make_async_copy(
                        bufs[i].at[pl.ds(t * _CHUNK, _CHUNK)],
                        acc.at[didx_sb.at[_TPB * q + t]],
                        ssem[i]).wait()

            def scale(q, i):
                buf = bufs[i]

                def grp(g, carry):
                    row = _TPB * q + g // 8
                    lane0 = (g % 8) * 16
                    v16 = vals_sb[row, pl.ds(lane0, 16)]
                    for j in range(16):
                        e = g * 16 + j
                        sv = v16[j]
                        buf[e, pl.ds(0, 16)] = buf[e, pl.ds(0, 16)] * sv
                        buf[e, pl.ds(16, 16)] = buf[e, pl.ds(16, 16)] * sv
                    return carry

                lax.fori_loop(0, _BLK // 16, grp, 0)

            def super_body(sb, carry):
                pltpu.async_copy(src2.at[pl.ds(sbase + sb * _SBR, _SBR)],
                                 sidx_sb, g2)
                pltpu.async_copy(dst2.at[pl.ds(dbase + sb * _SBR, _SBR)],
                                 didx_sb, g3)
                pltpu.async_copy(val2.at[pl.ds(dbase + sb * _SBR, _SBR)],
                                 vals_sb, s0)
                pltpu.make_async_copy(src2.at[pl.ds(sbase + sb * _SBR, _SBR)],
                                      sidx_sb, g2).wait()
                pltpu.make_async_copy(dst2.at[pl.ds(dbase + sb * _SBR, _SBR)],
                                      didx_sb, g3).wait()
                pltpu.make_async_copy(val2.at[pl.ds(dbase + sb * _SBR, _SBR)],
                                      vals_sb, s0).wait()
                gather_issue(0, 0)
                gather_issue(1, 1)

                def body(m, carry2):
                    for i in range(4):
                        q = 4 * m + i
                        j = (i + 2) % 4
                        gather_wait(q, i)
                        scale(q, i)
                        scat_issue(q, i)
                        if i < 2:
                            @pl.when(m > 0)
                            def _():
                                scat_wait(q - 2, j)

                            gather_issue(q + 2, j)
                        else:
                            @pl.when(m < _NBODY - 1)
                            def _():
                                scat_wait(q - 2, j)
                                gather_issue(q + 2, j)
                    return carry2

                lax.fori_loop(0, _NBODY, body, 0)
                for i in range(4):
                    scat_wait(_NBLK - 4 + i, i)
                return carry

            lax.fori_loop(0, _NSB, super_body, 0)

        process(srcA, dstA, valA, xA)
        process(srcB, dstB, valB, xB)
        plsc.subcore_barrier()

        def wb(k, carry):
            r0 = s * _RPT + k * _CHUNK
            pltpu.sync_copy(acc.at[pl.ds(r0, _CHUNK)],
                            out.at[pl.ds(c * _NP + r0, _CHUNK)])
            return carry

        lax.fori_loop(0, _RFULL, wb, 0)
        r0 = s * _RPT + _RFULL * _CHUNK
        pltpu.sync_copy(acc.at[pl.ds(r0, _RREM)],
                        out.at[pl.ds(c * _NP + r0, _RREM)])

    return spmm2


@functools.lru_cache(maxsize=None)
def _gamma():
    mesh = plsc.VectorSubcoreMesh(core_axis_name="c", subcore_axis_name="s")

    @functools.partial(
        pl.kernel,
        mesh=mesh,
        compiler_params=pltpu.CompilerParams(
            needs_layout_passes=False, use_tc_tiling_on_sc=False),
        out_type=jax.ShapeDtypeStruct((_B,), jnp.float32),
        scratch_types=[
            pltpu.VMEM((_CHUNK,), jnp.int32),
            pltpu.VMEM((_CHUNK,), jnp.int32),
            pltpu.VMEM((_CHUNK, _H), jnp.float32),
            pltpu.VMEM((_CHUNK, _H), jnp.float32),
            pltpu.VMEM((_CHUNK, _H), jnp.float32),
            pltpu.VMEM((_CHUNK, _H), jnp.float32),
            pltpu.VMEM((_BPT,), jnp.float32),
            pltpu.SemaphoreType.DMA,
        ],
    )
    def gk(au, ai, u0idx, u1idx, i0idx, i1idx, out,
           uidx, iidx, u0, u1, i0, i1, gv, sem):
        wid = lax.axis_index("s") * 2 + lax.axis_index("c")
        for kk in range(_BPT // _CHUNK):
            off = wid * _BPT + kk * _CHUNK
            pltpu.sync_copy(u0idx.at[pl.ds(off, _CHUNK)], uidx)
            pltpu.async_copy(au.at[uidx], u0, sem).wait()
            pltpu.sync_copy(u1idx.at[pl.ds(off, _CHUNK)], uidx)
            pltpu.async_copy(au.at[uidx], u1, sem).wait()
            pltpu.sync_copy(i0idx.at[pl.ds(off, _CHUNK)], iidx)
            pltpu.async_copy(ai.at[iidx], i0, sem).wait()
            pltpu.sync_copy(i1idx.at[pl.ds(off, _CHUNK)], iidx)
            pltpu.async_copy(ai.at[iidx], i1, sem).wait()

            def grp(g, carry):
                rowi = g * 16 + lax.iota(jnp.int32, 16)
                acc = jnp.zeros((16,), jnp.float32)
                for dd in range(_H):
                    cold = jnp.full((16,), dd, jnp.int32)
                    acc = acc + plsc.load_gather(u0, [rowi, cold]) * plsc.load_gather(i0, [rowi, cold])
                    acc = acc + plsc.load_gather(u1, [rowi, cold]) * plsc.load_gather(i1, [rowi, cold])
                gv[pl.ds(kk * _CHUNK + g * 16, 16)] = acc
                return carry

            lax.fori_loop(0, _CHUNK // 16, grp, 0)
        pltpu.sync_copy(gv, out.at[pl.ds(wid * _BPT, _BPT)])

    return gk


def _to2(w):
    wp = jnp.pad(w, ((0, _NP - w.shape[0]), (0, 0)))
    return jnp.concatenate([wp[:, :_H], wp[:, _H:]], axis=0)


def _pad_edges(src, dst, val):
    pz = _EPAD - _E
    srcp = jnp.concatenate([src, jnp.zeros((pz,), jnp.int32)])
    src2 = jnp.concatenate([srcp, srcp + _NP])
    dstp = jnp.concatenate([dst, jnp.zeros((pz,), jnp.int32)])
    valp = jnp.concatenate([val, jnp.zeros((pz,), jnp.float32)])
    return (src2.reshape(2 * _ROWS, _CHUNK),
            dstp.reshape(_ROWS, _CHUNK),
            valp.reshape(_ROWS, _CHUNK))


def kernel(users, items, uv_src, uv_dst, uv_val, uu_src, uu_dst, uu_val,
           vv_src, vv_dst, vv_val, vu_src, vu_dst, vu_val,
           du, dv, user_emb_w, item_emb_w):
    spmm2 = _spmm2()
    gk = _gamma()
    uw2, iw2 = _to2(user_emb_w), _to2(item_emb_w)
    duo, dvo = _to2(du * user_emb_w), _to2(dv * item_emb_w)
    uu = _pad_edges(uu_src, uu_dst, uu_val)
    uv = _pad_edges(uv_src, uv_dst, uv_val)
    vv = _pad_edges(vv_src, vv_dst, vv_val)
    vu = _pad_edges(vu_src, vu_dst, vu_val)
    ue, ie = uw2, iw2
    su, si = uw2, iw2
    for _ in range(3):
        ueff, ieff = ue + duo, ie + dvo
        ue = spmm2(uu[0], uu[1], uu[2], ueff, uv[0], uv[1], uv[2], ieff)
        ie = spmm2(vv[0], vv[1], vv[2], ieff, vu[0], vu[1], vu[2], ue)
        su, si = su + ue, si + ie
    au, ai = su * 0.25, si * 0.25
    return gk(au, ai, users, users + _NP, items, items + _NP)
